# baseline (device time: 56919 ns/iter reference)
import os

import jax
import jax.numpy as jnp
from jax import lax
from jax.experimental import pallas as pl
from jax.experimental.pallas import tpu as pltpu

W = 8
LOG_W = 3
OPS = [("pair", 0, 11), ("quad", 12), ("pair", 2, 13), ("quad", 13)]
OP_BASE = [0, 1, 4, 5]
TOT_W = 8
N_CHUNK = int(os.environ.get("SORT_CHUNKS", "4"))
NO_EXCH = bool(os.environ.get("SORT_NO_EXCH"))


def _cmpex(v, s, blk, offset):
    m, n = v.shape
    if s >= 16:
        grp = m // (2 * s)
        y = v.reshape(grp, 2, s, n)
        a, b = y[:, 0], y[:, 1]
        lo, hi = jnp.minimum(a, b), jnp.maximum(a, b)
        g0 = lax.broadcasted_iota(jnp.int32, (grp, 1, 1), 0) * (2 * s) + offset
        asc = (g0 & blk) == 0
        first = jnp.where(asc, lo, hi)
        second = jnp.where(asc, hi, lo)
        out = jnp.concatenate([first[:, None], second[:, None]], axis=1)
        return out.reshape(m, n)
    g = lax.broadcasted_iota(jnp.int32, v.shape, 0) + offset
    partner_above = (g & s) == 0
    down = jnp.concatenate([v[s:], v[:s]], axis=0)
    up = jnp.concatenate([v[m - s:], v[:m - s]], axis=0)
    partner = jnp.where(partner_above, down, up)
    asc = (g & blk) == 0
    take_min = partner_above == asc
    return jnp.where(take_min, jnp.minimum(v, partner), jnp.maximum(v, partner))


def kernel(x):
    m_per, n = x.shape
    n_half = n // 2
    m_chunk = m_per // N_CHUNK
    log_m_per = m_per.bit_length() - 1

    def body(x_ref, out_ref, sbufs, rbufs, send_sems, recv_sems):
        my = lax.axis_index("i")
        offset = my * m_per

        barrier_sem = pltpu.get_barrier_semaphore()
        for t in range(LOG_W):
            pl.semaphore_signal(
                barrier_sem, inc=1,
                device_id=(my ^ (1 << t),),
                device_id_type=pl.DeviceIdType.MESH,
            )
        pl.semaphore_wait(barrier_sem, LOG_W)

        def local_sort(v):
            for k in range(1, log_m_per + 1):
                for j in range(k - 1, -1, -1):
                    v = _cmpex(v, 1 << j, 1 << k, offset)
            return v

        def local_tail(v, k):
            for j in range(log_m_per - 1, -1, -1):
                v = _cmpex(v, 1 << j, 1 << k, offset)
            return v

        def slot(g, base, c):
            return (g * TOT_W + base) * N_CHUNK + c

        def rdma_to(partner, src, dst_slot):
            return pltpu.make_async_remote_copy(
                src_ref=src,
                dst_ref=rbufs.at[dst_slot],
                send_sem=send_sems.at[dst_slot],
                recv_sem=recv_sems.at[dst_slot],
                device_id=(partner,),
                device_id_type=pl.DeviceIdType.MESH,
            )

        def start_op(g, op, c, val):
            base = OP_BASE[op]
            sbufs[g, c] = val
            src = sbufs.at[g, c]
            if OPS[op][0] == "pair":
                rels = (1 << OPS[op][1],)
            else:
                rels = (1, 2, 3)
            handles = []
            for i, rel in enumerate(rels):
                r = rdma_to(my ^ rel, src, slot(g, base + i, c))
                if not NO_EXCH:
                    r.start()
                handles.append(r)
            return handles

        def finish_op(g, op, c, val, handles):
            if not NO_EXCH:
                for r in handles:
                    r.wait()
            base = OP_BASE[op]
            if OPS[op][0] == "pair":
                _, t, k = OPS[op]
                other = rbufs[slot(g, base, c)]
                asc = (offset & (1 << k)) == 0
                partner_above = (offset & (m_per << t)) == 0
                take_min = partner_above == asc
                return jnp.where(
                    take_min, jnp.minimum(val, other), jnp.maximum(val, other)
                )
            _, k = OPS[op]
            x1 = rbufs[slot(g, base + 0, c)]
            x2 = rbufs[slot(g, base + 1, c)]
            x3 = rbufs[slot(g, base + 2, c)]
            lo_p, hi_p = jnp.minimum(val, x2), jnp.maximum(val, x2)
            lo_q, hi_q = jnp.minimum(x1, x3), jnp.maximum(x1, x3)
            r0 = jnp.minimum(lo_p, lo_q)
            r1 = jnp.maximum(lo_p, lo_q)
            r2 = jnp.minimum(hi_p, hi_q)
            r3 = jnp.maximum(hi_p, hi_q)
            asc = (offset & (1 << k)) == 0
            q = my & 3
            qq = jnp.where(asc, q, 3 - q)
            return jnp.where(
                qq == 0, r0, jnp.where(qq == 1, r1, jnp.where(qq == 2, r2, r3))
            )

        def split(v):
            return [v[c * m_chunk:(c + 1) * m_chunk] for c in range(N_CHUNK)]

        def start_all(g, op):
            for c in range(N_CHUNK):
                rd[g][c] = start_op(g, op, c, chunks[g][c])

        def finish_all(g, op):
            for c in range(N_CHUNK):
                chunks[g][c] = finish_op(g, op, c, chunks[g][c], rd[g][c])

        chunks = [None, None]
        rd = [[None] * N_CHUNK, [None] * N_CHUNK]

        chunks[0] = split(local_sort(x_ref[:, :n_half].astype(jnp.bfloat16)))
        start_all(0, 0)
        chunks[1] = split(local_sort(x_ref[:, n_half:].astype(jnp.bfloat16)))
        start_all(1, 0)

        for g in (0, 1):
            finish_all(g, 0)
            chunks[g] = split(local_tail(jnp.concatenate(chunks[g], 0), 11))
            start_all(g, 1)
        for g in (0, 1):
            finish_all(g, 1)
            chunks[g] = split(local_tail(jnp.concatenate(chunks[g], 0), 12))
            start_all(g, 2)
        for g in (0, 1):
            for c in range(N_CHUNK):
                chunks[g][c] = finish_op(g, 2, c, chunks[g][c], rd[g][c])
                rd[g][c] = start_op(g, 3, c, chunks[g][c])
        for g in (0, 1):
            finish_all(g, 3)
            v = local_tail(jnp.concatenate(chunks[g], 0), 13)
            out_ref[:, pl.ds(g * n_half, n_half)] = v.astype(out_ref.dtype)

    return pl.pallas_call(
        body,
        out_shape=jax.ShapeDtypeStruct((m_per, n), x.dtype),
        in_specs=[pl.BlockSpec(memory_space=pltpu.VMEM)],
        out_specs=pl.BlockSpec(memory_space=pltpu.VMEM),
        scratch_shapes=[
            pltpu.VMEM((2, N_CHUNK, m_chunk, n_half), jnp.bfloat16),
            pltpu.VMEM((2 * TOT_W * N_CHUNK, m_chunk, n_half), jnp.bfloat16),
            pltpu.SemaphoreType.DMA((2 * TOT_W * N_CHUNK,)),
            pltpu.SemaphoreType.DMA((2 * TOT_W * N_CHUNK,)),
        ],
        compiler_params=pltpu.CompilerParams(
            collective_id=0,
            vmem_limit_bytes=100 * 1024 * 1024,
        ),
    )(x)


# device time: 55590 ns/iter; 1.0239x vs baseline; 1.0239x over previous
import os

import jax
import jax.numpy as jnp
from jax import lax
from jax.experimental import pallas as pl
from jax.experimental.pallas import tpu as pltpu

W = 8
LOG_W = 3
PLAN = [(11, 0), (12, 1), (12, 0), (13, 2), (13, 1), (13, 0)]
TAIL_AFTER = {0: 11, 2: 12, 5: 13}
N_EXCH = len(PLAN)
N_CHUNK = int(os.environ.get("SORT_CHUNKS", "4"))
NO_EXCH = bool(os.environ.get("SORT_NO_EXCH"))


def _cmpex(v, s, blk, offset):
    m, n = v.shape
    if s >= 16:
        grp = m // (2 * s)
        y = v.reshape(grp, 2, s, n)
        a, b = y[:, 0], y[:, 1]
        lo, hi = jnp.minimum(a, b), jnp.maximum(a, b)
        g0 = lax.broadcasted_iota(jnp.int32, (grp, 1, 1), 0) * (2 * s) + offset
        asc = (g0 & blk) == 0
        first = jnp.where(asc, lo, hi)
        second = jnp.where(asc, hi, lo)
        out = jnp.concatenate([first[:, None], second[:, None]], axis=1)
        return out.reshape(m, n)
    g = lax.broadcasted_iota(jnp.int32, v.shape, 0) + offset
    partner_above = (g & s) == 0
    if os.environ.get("SORT_ROLL"):
        down = pltpu.roll(v, m - s, 0)
        up = pltpu.roll(v, s, 0)
    else:
        down = jnp.concatenate([v[s:], v[:s]], axis=0)
        up = jnp.concatenate([v[m - s:], v[:m - s]], axis=0)
    partner = jnp.where(partner_above, down, up)
    asc = (g & blk) == 0
    take_min = partner_above == asc
    return jnp.where(take_min, jnp.minimum(v, partner), jnp.maximum(v, partner))


def kernel(x):
    m_per, n = x.shape
    n_half = n // 2
    m_chunk = m_per // N_CHUNK
    log_m_per = m_per.bit_length() - 1

    def body(x_ref, out_ref, sbufs, rbufs, send_sems, recv_sems):
        my = lax.axis_index("i")
        offset = my * m_per

        barrier_sem = pltpu.get_barrier_semaphore()
        for t in range(LOG_W):
            pl.semaphore_signal(
                barrier_sem, inc=1,
                device_id=(my ^ (1 << t),),
                device_id_type=pl.DeviceIdType.MESH,
            )
        pl.semaphore_wait(barrier_sem, LOG_W)

        def local_sort(v):
            for k in range(1, log_m_per + 1):
                for j in range(k - 1, -1, -1):
                    v = _cmpex(v, 1 << j, 1 << k, offset)
            return v

        def local_tail(v, k):
            for j in range(log_m_per - 1, -1, -1):
                v = _cmpex(v, 1 << j, 1 << k, offset)
            return v

        def start_half(g, e, h, val):
            _, t = PLAN[e]
            slot = (g * N_EXCH + e) * N_CHUNK + h
            sbufs[g, h] = val
            rdma = pltpu.make_async_remote_copy(
                src_ref=sbufs.at[g, h],
                dst_ref=rbufs.at[slot],
                send_sem=send_sems.at[slot],
                recv_sem=recv_sems.at[slot],
                device_id=(my ^ (1 << t),),
                device_id_type=pl.DeviceIdType.MESH,
            )
            if not NO_EXCH:
                rdma.start()
            return rdma

        def finish_half(g, e, h, val, rdma):
            if not NO_EXCH:
                rdma.wait()
            k, t = PLAN[e]
            other = rbufs[(g * N_EXCH + e) * N_CHUNK + h]
            asc = (offset & (1 << k)) == 0
            partner_above = (offset & (m_per << t)) == 0
            take_min = partner_above == asc
            return jnp.where(
                take_min, jnp.minimum(val, other), jnp.maximum(val, other)
            )

        def split(v):
            return [v[c * m_chunk:(c + 1) * m_chunk] for c in range(N_CHUNK)]

        chunks = [None, None]
        rd = [[None] * N_CHUNK, [None] * N_CHUNK]
        vA = local_sort(x_ref[:, :n_half].astype(jnp.bfloat16))
        chunks[0] = split(vA)
        for h in range(N_CHUNK):
            rd[0][h] = start_half(0, 0, h, chunks[0][h])
        vB = local_sort(x_ref[:, n_half:].astype(jnp.bfloat16))
        chunks[1] = split(vB)
        for h in range(N_CHUNK):
            rd[1][h] = start_half(1, 0, h, chunks[1][h])

        for e in range(N_EXCH):
            nxt = e + 1
            for g in (0, 1):
                if e in TAIL_AFTER:
                    for h in range(N_CHUNK):
                        chunks[g][h] = finish_half(g, e, h, chunks[g][h], rd[g][h])
                    v = jnp.concatenate(chunks[g], axis=0)
                    v = local_tail(v, TAIL_AFTER[e])
                    if nxt < N_EXCH:
                        chunks[g] = split(v)
                        for h in range(N_CHUNK):
                            rd[g][h] = start_half(g, nxt, h, chunks[g][h])
                    else:
                        cols = pl.ds(g * n_half, n_half)
                        out_ref[:, cols] = v.astype(out_ref.dtype)
                else:
                    for h in range(N_CHUNK):
                        chunks[g][h] = finish_half(g, e, h, chunks[g][h], rd[g][h])
                        rd[g][h] = start_half(g, nxt, h, chunks[g][h])

    return pl.pallas_call(
        body,
        out_shape=jax.ShapeDtypeStruct((m_per, n), x.dtype),
        in_specs=[pl.BlockSpec(memory_space=pltpu.VMEM)],
        out_specs=pl.BlockSpec(memory_space=pltpu.VMEM),
        scratch_shapes=[
            pltpu.VMEM((2, N_CHUNK, m_chunk, n_half), jnp.bfloat16),
            pltpu.VMEM((2 * N_EXCH * N_CHUNK, m_chunk, n_half), jnp.bfloat16),
            pltpu.SemaphoreType.DMA((2 * N_EXCH * N_CHUNK,)),
            pltpu.SemaphoreType.DMA((2 * N_EXCH * N_CHUNK,)),
        ],
        compiler_params=pltpu.CompilerParams(
            collective_id=0,
            vmem_limit_bytes=100 * 1024 * 1024,
        ),
    )(x)
